# baseline (speedup 1.0000x reference)
import jax
import jax.numpy as jnp
from jax import lax
from jax.experimental import pallas as pl
from jax.experimental.pallas import tpu as pltpu

_B = 4096
_CHUNKS = [(0, 840), (840, 840), (1680, 840), (2520, 821)]

_CT = (((1,), (0,)), ((), ()))
_CTT = (((0,), (0,)), ((), ()))


def _w0_copies(w0_any, w0_v, sem):
    return [pltpu.make_async_copy(w0_any.at[pl.ds(s, n), :],
                                  w0_v.at[pl.ds(s, n), :], sem)
            for s, n in _CHUNKS]


def _mlp_kernel(numt_ref, emb_ref, w0t16_ref, b0_ref,
                w1_ref, b1_ref, w2_ref, b2_ref, w3t_ref, b3_ref,
                w0_any,
                out_ref,
                p_ref, c0_ref, w1b_ref, w2b_ref, w3b_ref, w0_v, sem):
    bf = jnp.bfloat16
    n_num = numt_ref.shape[0]
    n_tab = emb_ref.shape[0]
    g = pl.program_id(0)

    @pl.when(g == 0)
    def _():
        for c in _w0_copies(w0_any, w0_v, sem):
            c.start()
        xt = numt_ref[...].astype(bf)
        w0t = w0t16_ref[0:n_num, :].astype(bf)
        p_ref[...] = lax.dot_general(w0t, xt, _CTT,
                                     preferred_element_type=jnp.float32)

    @pl.when(g == 1)
    def _():
        for c in _w0_copies(w0_any, w0_v, sem):
            c.wait()
        parts = [jnp.zeros((1, n_num), jnp.float32)]
        parts += [emb_ref[i, :, :] for i in range(n_tab)]
        emb_pad = jnp.concatenate(parts, axis=1)
        c0 = jnp.dot(emb_pad, w0_v[...],
                     preferred_element_type=jnp.float32)
        c0_ref[...] = (c0 + b0_ref[...]).T
        w1b_ref[...] = w1_ref[...].astype(bf)
        w2b_ref[...] = w2_ref[...].astype(bf)
        w3b_ref[...] = w3t_ref[...].astype(bf)
        b1c = b1_ref[...].T
        b2c = b2_ref[...].T

        h = jnp.maximum(p_ref[...] + c0_ref[...], 0.0).astype(bf)
        h = jnp.maximum(
            lax.dot_general(w1b_ref[...], h, _CTT,
                            preferred_element_type=jnp.float32)
            + b1c, 0.0).astype(bf)
        h = jnp.maximum(
            lax.dot_general(w2b_ref[...], h, _CTT,
                            preferred_element_type=jnp.float32)
            + b2c, 0.0).astype(bf)
        o = lax.dot_general(w3b_ref[...], h, _CT,
                            preferred_element_type=jnp.float32)
        out_ref[...] = (o + b3_ref[...])[0, :]


def kernel(numerical_features, categorical_features, emb_tables,
           W0, b0, W1, b1, W2, b2, W3, b3):
    del categorical_features
    n_num = numerical_features.shape[1]

    const = lambda i: (0, 0)
    const3 = lambda i: (0, 0, 0)
    out = pl.pallas_call(
        _mlp_kernel,
        grid=(2,),
        in_specs=[
            pl.BlockSpec((n_num, _B), const),
            pl.BlockSpec(emb_tables.shape, const3),
            pl.BlockSpec((16, 1024), const),
            pl.BlockSpec((1, b0.shape[0]), const),
            pl.BlockSpec(W1.shape, const),
            pl.BlockSpec((1, b1.shape[0]), const),
            pl.BlockSpec(W2.shape, const),
            pl.BlockSpec((1, b2.shape[0]), const),
            pl.BlockSpec((1, W3.shape[0]), const),
            pl.BlockSpec((1, 1), const),
            pl.BlockSpec(memory_space=pltpu.MemorySpace.HBM),
        ],
        out_specs=pl.BlockSpec((_B,), lambda i: (0,)),
        out_shape=jax.ShapeDtypeStruct((_B,), jnp.float32),
        scratch_shapes=[
            pltpu.VMEM((b0.shape[0], _B), jnp.float32),
            pltpu.VMEM((b0.shape[0], 1), jnp.float32),
            pltpu.VMEM(W1.shape, jnp.bfloat16),
            pltpu.VMEM(W2.shape, jnp.bfloat16),
            pltpu.VMEM((1, W3.shape[0]), jnp.bfloat16),
            pltpu.VMEM(W0.shape, jnp.float32),
            pltpu.SemaphoreType.DMA,
        ],
    )(numerical_features.T, emb_tables, W0, b0.reshape(1, -1),
      W1, b1.reshape(1, -1), W2, b2.reshape(1, -1), W3.T, b3.reshape(1, 1),
      W0)
    return out
